# baseline (speedup 1.0000x reference)
import jax
import jax.numpy as jnp
from jax.experimental import pallas as pl
from jax.experimental.pallas import tpu as pltpu

_OUT_C = 384
_W = 0.5


def _scale_copy(perm_ref, x_ref, o_ref):
    o_ref[...] = x_ref[...] * _W


def kernel(x, perm):
    B, C, H, W = x.shape
    HW = H * W
    xr = x.reshape(B, C, HW // 128, 128)
    perm32 = perm.astype(jnp.int32)

    out = pl.pallas_call(
        _scale_copy,
        grid_spec=pltpu.PrefetchScalarGridSpec(
            num_scalar_prefetch=1,
            grid=(_OUT_C,),
            in_specs=[
                pl.BlockSpec(
                    (B, 1, HW // 128, 128),
                    lambda j, perm_ref: (0, perm_ref[j], 0, 0),
                )
            ],
            out_specs=pl.BlockSpec(
                (B, 1, HW // 128, 128),
                lambda j, perm_ref: (0, j, 0, 0),
            ),
        ),
        out_shape=jax.ShapeDtypeStruct((B, _OUT_C, HW // 128, 128), x.dtype),
    )(perm32, xr)
    return out.reshape(B, _OUT_C, H, W)


# speedup vs baseline: 1.2185x; 1.2185x over previous
import jax
import jax.numpy as jnp
from jax.experimental import pallas as pl
from jax.experimental.pallas import tpu as pltpu

_IN_C = 192
_OUT_C = 384
_W = 0.5


def _body(dest_ref, x_ref, out_ref, scratch, sem):
    i = pl.program_id(0)
    slot = jax.lax.rem(i, 2)

    def _copies(step, s):
        d0 = dest_ref[step]
        d1 = dest_ref[_IN_C + step]
        c0 = pltpu.make_async_copy(
            scratch.at[s], out_ref.at[:, pl.ds(d0, 1)], sem.at[s, 0]
        )
        c1 = pltpu.make_async_copy(
            scratch.at[s], out_ref.at[:, pl.ds(d1, 1)], sem.at[s, 1]
        )
        return c0, c1

    @pl.when(i >= 2)
    def _():
        c0, c1 = _copies(i - 2, slot)
        c0.wait()
        c1.wait()

    scratch[slot] = x_ref[...] * _W

    c0, c1 = _copies(i, slot)
    c0.start()
    c1.start()

    @pl.when(i == _IN_C - 1)
    def _():
        p0, p1 = _copies(i - 1, 1 - slot)
        p0.wait()
        p1.wait()
        c0, c1 = _copies(i, slot)
        c0.wait()
        c1.wait()


def kernel(x, perm):
    B, C, H, W = x.shape
    HW = H * W
    S = HW // 128
    xr = x.reshape(B, C, S, 128)

    perm32 = perm.astype(jnp.int32)
    ar = jnp.arange(_IN_C, dtype=jnp.int32)
    z = jnp.zeros((_IN_C,), jnp.int32)
    dest0 = z.at[perm32[:_IN_C]].set(ar)
    dest1 = z.at[perm32[_IN_C:]].set(ar) + _IN_C
    dests = jnp.concatenate([dest0, dest1])

    out = pl.pallas_call(
        _body,
        grid_spec=pltpu.PrefetchScalarGridSpec(
            num_scalar_prefetch=1,
            grid=(_IN_C,),
            in_specs=[
                pl.BlockSpec((B, 1, S, 128), lambda i, dest_ref: (0, i, 0, 0))
            ],
            out_specs=pl.BlockSpec(memory_space=pl.MemorySpace.ANY),
            scratch_shapes=[
                pltpu.VMEM((2, B, 1, S, 128), jnp.float32),
                pltpu.SemaphoreType.DMA((2, 2)),
            ],
        ),
        out_shape=jax.ShapeDtypeStruct((B, _OUT_C, S, 128), x.dtype),
    )(dests, xr)
    return out.reshape(B, _OUT_C, H, W)
